# baseline (speedup 1.0000x reference)
import jax
import jax.numpy as jnp
from jax.experimental import pallas as pl


def kernel(x, edge_index, mask_nodes, enc_mask_token, W_enc1, b_enc1, W_enc2, b_enc2, W_e2d, W_dec, b_dec):
    N = x.shape[0]

    def _gcn(h, src, dst, W, b, relu, norm_out, norm_in):
        h = h * norm_out[:, None]
        h = h @ W
        agg = jnp.zeros((N, h.shape[1]), jnp.float32).at[dst].add(h[src])
        agg = agg * norm_in[:, None]
        out = agg + b
        if relu:
            out = jax.nn.relu(out)
        return out

    src, dst = edge_index[0], edge_index[1]
    deg_out = jnp.zeros((N,), jnp.float32).at[src].add(1.0)
    deg_in = jnp.zeros((N,), jnp.float32).at[dst].add(1.0)
    norm_out = jnp.where(deg_out > 0, deg_out ** -0.5, 0.0)
    norm_in = jnp.where(deg_in > 0, deg_in ** -0.5, 0.0)
    out_x = x.at[mask_nodes].set(0.0)
    out_x = out_x.at[mask_nodes].add(enc_mask_token[0])
    h = _gcn(out_x, src, dst, W_enc1, b_enc1, True, norm_out, norm_in)
    enc_rep = _gcn(h, src, dst, W_enc2, b_enc2, True, norm_out, norm_in)
    rep = enc_rep @ W_e2d
    rep = rep.at[mask_nodes].set(0.0)
    recon = _gcn(rep, src, dst, W_dec, b_dec, False, norm_out, norm_in)
    x_init = x[mask_nodes]
    x_rec = recon[mask_nodes]
    xn = x_rec / jnp.maximum(jnp.linalg.norm(x_rec, axis=-1, keepdims=True), 1e-12)
    yn = x_init / jnp.maximum(jnp.linalg.norm(x_init, axis=-1, keepdims=True), 1e-12)
    loss = jnp.mean((1.0 - jnp.sum(xn * yn, axis=-1)) ** 2)
    return (enc_rep, recon, loss)


# speedup vs baseline: 4.7538x; 4.7538x over previous
import functools

import jax
import jax.numpy as jnp
from jax import lax
from jax.experimental import pallas as pl
from jax.experimental.pallas import tpu as pltpu
from jax.experimental.pallas import tpu_sc as plsc

NC = 2
NS = 16
MB = 128

_MESH = plsc.VectorSubcoreMesh(
    core_axis_name="c", subcore_axis_name="s", num_cores=NC, num_subcores=NS)


def _sc_degrees(edge_index, zpad, ones128, NP):
    E = edge_index.shape[1]
    nmb = E // MB
    nmb_core = nmb // NC
    rows_t = NP // NS

    @functools.partial(
        pl.kernel,
        out_type=jax.ShapeDtypeStruct((NC, 2, NP, 128), jnp.float32),
        mesh=_MESH,
        scratch_types=[
            pltpu.VMEM((MB,), jnp.int32),
            pltpu.VMEM((MB, 128), jnp.float32),
            pltpu.VMEM_SHARED((NP, 128), jnp.float32),
        ],
    )
    def deg_kernel(ei, zp, ones_hbm, out, idx_v, ones_v, acc):
        c = lax.axis_index("c")
        s = lax.axis_index("s")
        r0 = s * rows_t
        pltpu.sync_copy(ones_hbm, ones_v)
        nit = (nmb_core - s + NS - 1) // NS

        def run_pass(which):
            pltpu.sync_copy(zp.at[pl.ds(r0, rows_t), :],
                            acc.at[pl.ds(r0, rows_t), :])
            plsc.subcore_barrier()

            def body(i, _):
                m = c * nmb_core + i * NS + s
                base = m * MB
                pltpu.sync_copy(ei.at[which, pl.ds(base, MB)], idx_v)
                pltpu.sync_copy(ones_v, acc.at[idx_v], add=True)
                return 0

            lax.fori_loop(0, nit, body, 0)
            plsc.subcore_barrier()
            pltpu.sync_copy(acc.at[pl.ds(r0, rows_t), :],
                            out.at[c, which, pl.ds(r0, rows_t), :])

        run_pass(0)
        plsc.subcore_barrier()
        run_pass(1)

    return deg_kernel(edge_index, zpad, ones128)


def _sc_agg128(table, edge_index, zpad, NP):
    E = edge_index.shape[1]
    nmb = E // MB
    nmb_core = nmb // NC
    rows_t = NP // NS

    @functools.partial(
        pl.kernel,
        out_type=jax.ShapeDtypeStruct((NC, NP, 128), jnp.float32),
        mesh=_MESH,
        scratch_types=[
            pltpu.VMEM((MB,), jnp.int32),
            pltpu.VMEM((MB,), jnp.int32),
            pltpu.VMEM((MB, 128), jnp.float32),
            pltpu.VMEM_SHARED((NP, 128), jnp.float32),
            pltpu.SemaphoreType.DMA,
        ],
    )
    def agg_kernel(tab, ei, zp, out, sidx, didx, rows_v, acc, sem):
        c = lax.axis_index("c")
        s = lax.axis_index("s")
        r0 = s * rows_t
        pltpu.sync_copy(zp.at[pl.ds(r0, rows_t), :],
                        acc.at[pl.ds(r0, rows_t), :])
        plsc.subcore_barrier()

        nit = (nmb_core - s + NS - 1) // NS

        def body(i, _):
            m = c * nmb_core + i * NS + s
            base = m * MB
            pltpu.sync_copy(ei.at[0, pl.ds(base, MB)], sidx)
            pltpu.async_copy(tab.at[sidx], rows_v, sem).wait()
            pltpu.sync_copy(ei.at[1, pl.ds(base, MB)], didx)
            pltpu.sync_copy(rows_v, acc.at[didx], add=True)
            return 0

        lax.fori_loop(0, nit, body, 0)
        plsc.subcore_barrier()
        pltpu.sync_copy(acc.at[pl.ds(r0, rows_t), :],
                        out.at[c, pl.ds(r0, rows_t), :])

    return agg_kernel(table, edge_index, zpad)


def _sc_agg512(t0, t1, t2, t3, edge_index, zpad, NP):
    E = edge_index.shape[1]
    nmb = E // MB
    rows_t = NP // NS
    ot = jax.ShapeDtypeStruct((NP, 128), jnp.float32)

    @functools.partial(
        pl.kernel,
        out_type=(ot, ot, ot, ot),
        mesh=_MESH,
        scratch_types=[
            pltpu.VMEM((MB,), jnp.int32),
            pltpu.VMEM((MB,), jnp.int32),
            pltpu.VMEM((MB, 128), jnp.float32),
            pltpu.VMEM_SHARED((NP, 128), jnp.float32),
            pltpu.SemaphoreType.DMA,
        ],
    )
    def agg_kernel(a0, a1, a2, a3, ei, zp, o0, o1, o2, o3,
                   sidx, didx, rows_v, acc, sem):
        c = lax.axis_index("c")
        s = lax.axis_index("s")
        r0 = s * rows_t
        nit = (nmb - s + NS - 1) // NS

        def run_chunk(tab, out):
            pltpu.sync_copy(zp.at[pl.ds(r0, rows_t), :],
                            acc.at[pl.ds(r0, rows_t), :])
            plsc.subcore_barrier()

            def body(i, _):
                base = (i * NS + s) * MB
                pltpu.sync_copy(ei.at[0, pl.ds(base, MB)], sidx)
                pltpu.async_copy(tab.at[sidx], rows_v, sem).wait()
                pltpu.sync_copy(ei.at[1, pl.ds(base, MB)], didx)
                pltpu.sync_copy(rows_v, acc.at[didx], add=True)
                return 0

            lax.fori_loop(0, nit, body, 0)
            plsc.subcore_barrier()
            pltpu.sync_copy(acc.at[pl.ds(r0, rows_t), :],
                            out.at[pl.ds(r0, rows_t), :])

        @pl.when(c == 0)
        def _():
            run_chunk(a0, o0)
            plsc.subcore_barrier()
            run_chunk(a1, o1)

        @pl.when(c == 1)
        def _():
            run_chunk(a2, o2)
            plsc.subcore_barrier()
            run_chunk(a3, o3)

    return agg_kernel(t0, t1, t2, t3, edge_index, zpad)


def _tc_prep(x_pad, degp, maskcol, token, NP, MBK):
    grid = NP // MBK

    def body(x_ref, deg_ref, m_ref, tok_ref, oxn_ref, ni_ref, no_ref, mns_ref):
        dego = deg_ref[0, 0, :, 0:1] + deg_ref[1, 0, :, 0:1]
        degi = deg_ref[0, 1, :, 0:1] + deg_ref[1, 1, :, 0:1]
        no = jnp.where(dego > 0, lax.rsqrt(jnp.maximum(dego, 1e-30)), 0.0)
        ni = jnp.where(degi > 0, lax.rsqrt(jnp.maximum(degi, 1e-30)), 0.0)
        m = m_ref[...]
        ox = x_ref[...] * m + (1.0 - m) * tok_ref[...]
        oxn_ref[...] = ox * no
        ni_ref[...] = ni
        no_ref[...] = no
        mns_ref[...] = m * no

    return pl.pallas_call(
        body,
        grid=(grid,),
        in_specs=[
            pl.BlockSpec((MBK, 128), lambda i: (i, 0)),
            pl.BlockSpec((2, 2, MBK, 128), lambda i: (0, 0, i, 0)),
            pl.BlockSpec((MBK, 1), lambda i: (i, 0)),
            pl.BlockSpec((1, 128), lambda i: (0, 0)),
        ],
        out_specs=[
            pl.BlockSpec((MBK, 128), lambda i: (i, 0)),
            pl.BlockSpec((MBK, 1), lambda i: (i, 0)),
            pl.BlockSpec((MBK, 1), lambda i: (i, 0)),
            pl.BlockSpec((MBK, 1), lambda i: (i, 0)),
        ],
        out_shape=[
            jax.ShapeDtypeStruct((NP, 128), jnp.float32),
            jax.ShapeDtypeStruct((NP, 1), jnp.float32),
            jax.ShapeDtypeStruct((NP, 1), jnp.float32),
            jax.ShapeDtypeStruct((NP, 1), jnp.float32),
        ],
    )(x_pad, degp, maskcol, token)


def _tc_wed(W_e2d, W_dec):
    def body(a_ref, b_ref, o_ref):
        o_ref[...] = jnp.dot(a_ref[...], b_ref[...],
                             preferred_element_type=jnp.float32)

    return pl.pallas_call(
        body,
        out_shape=jax.ShapeDtypeStruct((512, 128), jnp.float32),
    )(W_e2d, W_dec)


def _tc_layer1(agg1, W1, b1, normin, normout, NP, MBK):
    grid = (NP // MBK, 4)

    def body(a_ref, w_ref, b_ref, ni_ref, no_ref, o_ref):
        a = a_ref[0] + a_ref[1]
        acc = jnp.dot(a, w_ref[...], preferred_element_type=jnp.float32)
        h = jnp.maximum(acc * ni_ref[...] + b_ref[...], 0.0)
        o_ref[0] = h * no_ref[...]

    return pl.pallas_call(
        body,
        grid=grid,
        in_specs=[
            pl.BlockSpec((2, MBK, 128), lambda i, c: (0, i, 0)),
            pl.BlockSpec((128, 128), lambda i, c: (0, c)),
            pl.BlockSpec((1, 128), lambda i, c: (0, c)),
            pl.BlockSpec((MBK, 1), lambda i, c: (i, 0)),
            pl.BlockSpec((MBK, 1), lambda i, c: (i, 0)),
        ],
        out_specs=pl.BlockSpec((1, MBK, 128), lambda i, c: (c, i, 0)),
        out_shape=jax.ShapeDtypeStruct((4, NP, 128), jnp.float32),
    )(agg1, W1, b1, normin, normout)


def _tc_layer2(agg2, W2, b2, normin, mns, W_ed, NP, MBK):
    grid = (NP // MBK,)

    def body(a_ref, w_ref, b_ref, ni_ref, mns_ref, wed_ref, enc_ref, d_ref):
        acc = jnp.dot(a_ref[0], w_ref[pl.ds(0, 128), :],
                      preferred_element_type=jnp.float32)
        for cc in range(1, 4):
            acc += jnp.dot(a_ref[cc], w_ref[pl.ds(cc * 128, 128), :],
                           preferred_element_type=jnp.float32)
        enc = jnp.maximum(acc * ni_ref[...] + b_ref[...], 0.0)
        enc_ref[...] = enc
        d_ref[...] = jnp.dot(enc * mns_ref[...], wed_ref[...],
                             preferred_element_type=jnp.float32)

    return pl.pallas_call(
        body,
        grid=grid,
        in_specs=[
            pl.BlockSpec((4, MBK, 128), lambda i: (0, i, 0)),
            pl.BlockSpec((512, 512), lambda i: (0, 0)),
            pl.BlockSpec((1, 512), lambda i: (0, 0)),
            pl.BlockSpec((MBK, 1), lambda i: (i, 0)),
            pl.BlockSpec((MBK, 1), lambda i: (i, 0)),
            pl.BlockSpec((512, 128), lambda i: (0, 0)),
        ],
        out_specs=[
            pl.BlockSpec((MBK, 512), lambda i: (i, 0)),
            pl.BlockSpec((MBK, 128), lambda i: (i, 0)),
        ],
        out_shape=[
            jax.ShapeDtypeStruct((NP, 512), jnp.float32),
            jax.ShapeDtypeStruct((NP, 128), jnp.float32),
        ],
    )(agg2, W2, b2, normin, mns, W_ed)


def _tc_final(agg3, b_dec, normin, maskcol, x_pad, NP, MBK):
    grid = (NP // MBK,)

    def body(a_ref, b_ref, ni_ref, m_ref, x_ref, rec_ref, loss_ref):
        i = pl.program_id(0)
        r = (a_ref[0] + a_ref[1]) * ni_ref[...] + b_ref[...]
        rec_ref[...] = r
        w = 1.0 - m_ref[...]
        x = x_ref[...]
        rnorm = jnp.sqrt(jnp.sum(r * r, axis=-1, keepdims=True))
        xnorm = jnp.sqrt(jnp.sum(x * x, axis=-1, keepdims=True))
        rn = r / jnp.maximum(rnorm, 1e-12)
        xn = x / jnp.maximum(xnorm, 1e-12)
        cos = jnp.sum(rn * xn, axis=-1, keepdims=True)
        contrib = jnp.sum(w * (1.0 - cos) ** 2, keepdims=True).reshape(1, 1)

        @pl.when(i == 0)
        def _():
            loss_ref[...] = contrib

        @pl.when(i > 0)
        def _():
            loss_ref[...] += contrib

    return pl.pallas_call(
        body,
        grid=grid,
        in_specs=[
            pl.BlockSpec((2, MBK, 128), lambda i: (0, i, 0)),
            pl.BlockSpec((1, 128), lambda i: (0, 0)),
            pl.BlockSpec((MBK, 1), lambda i: (i, 0)),
            pl.BlockSpec((MBK, 1), lambda i: (i, 0)),
            pl.BlockSpec((MBK, 128), lambda i: (i, 0)),
        ],
        out_specs=[
            pl.BlockSpec((MBK, 128), lambda i: (i, 0)),
            pl.BlockSpec((1, 1), lambda i: (0, 0)),
        ],
        out_shape=[
            jax.ShapeDtypeStruct((NP, 128), jnp.float32),
            jax.ShapeDtypeStruct((1, 1), jnp.float32),
        ],
    )(agg3, b_dec, normin, maskcol, x_pad)


def kernel(x, edge_index, mask_nodes, enc_mask_token,
           W_enc1, b_enc1, W_enc2, b_enc2, W_e2d, W_dec, b_dec):
    N = x.shape[0]
    num_mask = mask_nodes.shape[0]
    NP = ((N + NS * 40 - 1) // (NS * 40)) * (NS * 40)
    MBK = NP // 8

    x_pad = jnp.pad(x, ((0, NP - N), (0, 0)))
    maskcol = jnp.ones((NP, 1), jnp.float32).at[mask_nodes].set(0.0)
    zpad = jnp.zeros((NP, 128), jnp.float32)
    ones128 = jnp.ones((MB, 128), jnp.float32)

    degp = _sc_degrees(edge_index, zpad, ones128, NP)
    oxn, normin, normout, mns = _tc_prep(
        x_pad, degp, maskcol, enc_mask_token, NP, MBK)
    W_ed = _tc_wed(W_e2d, W_dec)

    agg1 = _sc_agg128(oxn, edge_index, zpad, NP)
    h1n = _tc_layer1(agg1, W_enc1, b_enc1.reshape(1, -1), normin, normout,
                     NP, MBK)
    agg2c = _sc_agg512(h1n[0], h1n[1], h1n[2], h1n[3], edge_index, zpad, NP)
    agg2 = jnp.stack(agg2c)
    enc_pad, d = _tc_layer2(agg2, W_enc2, b_enc2.reshape(1, -1), normin, mns,
                            W_ed, NP, MBK)
    agg3 = _sc_agg128(d, edge_index, zpad, NP)
    recon_pad, loss_acc = _tc_final(agg3, b_dec.reshape(1, -1), normin,
                                    maskcol, x_pad, NP, MBK)

    enc_rep = enc_pad[:N]
    recon = recon_pad[:N]
    loss = (loss_acc[0, 0] / num_mask).astype(jnp.float32)
    return (enc_rep, recon, loss)
